# Initial kernel scaffold; baseline (speedup 1.0000x reference)
import jax
import jax.numpy as jnp
from jax.experimental import pallas as pl


def kernel(table, W1, b1, W2, b2, Wc1, bc1, Wc2, bc2, context_indices, candidate_indices):
    raise NotImplementedError("write your pallas kernel here")


# speedup vs baseline: 1.1339x; 1.1339x over previous
import functools

import jax
import jax.numpy as jnp
from jax import lax
from jax.experimental import pallas as pl
from jax.experimental.pallas import tpu as pltpu
from jax.experimental.pallas import tpu_sc as plsc

EMB = 64
CTX_PAD = 56
CAND_PAD = 312
NW = 32


def _sc_pool(table, ctx_idx_flat, cand_idx_flat, B, C, LCTX, LC):
    SPW = B // NW
    mesh = plsc.VectorSubcoreMesh(core_axis_name="c", subcore_axis_name="s")

    @functools.partial(
        pl.kernel,
        mesh=mesh,
        compiler_params=pltpu.CompilerParams(use_tc_tiling_on_sc=False),
        out_type=[
            jax.ShapeDtypeStruct((B * C * EMB,), jnp.float32),
            jax.ShapeDtypeStruct((B, EMB), jnp.float32),
        ],
        scratch_types=[
            pltpu.VMEM((SPW * CTX_PAD,), jnp.int32),
            pltpu.VMEM((SPW * CAND_PAD,), jnp.int32),
            pltpu.VMEM((CTX_PAD, EMB), jnp.float32),
            pltpu.VMEM((CAND_PAD, EMB), jnp.float32),
            pltpu.VMEM((C * EMB,), jnp.float32),
            pltpu.VMEM((SPW, EMB), jnp.float32),
            pltpu.SemaphoreType.DMA,
        ],
    )
    def sc_kernel(table_hbm, ctx_idx_hbm, cand_idx_hbm,
                  comb_hbm, ctxe_hbm,
                  ctx_idx_v, cand_idx_v, ctx_rows_v, cand_rows_v,
                  comb_v, ctxe_v, sem):
        wid = lax.axis_index("s") * 2 + lax.axis_index("c")
        s0 = wid * SPW

        pltpu.sync_copy(ctx_idx_hbm.at[pl.ds(s0 * CTX_PAD, SPW * CTX_PAD)],
                        ctx_idx_v)
        pltpu.sync_copy(cand_idx_hbm.at[pl.ds(s0 * CAND_PAD, SPW * CAND_PAD)],
                        cand_idx_v)

        inv_ctx = jnp.float32(1.0 / LCTX)
        inv_lc = jnp.float32(1.0 / LC)
        zero = jnp.zeros((16,), jnp.float32)

        def sample_body(s, carry):
            d0 = pltpu.async_copy(
                table_hbm.at[ctx_idx_v.at[pl.ds(s * CTX_PAD, CTX_PAD)]],
                ctx_rows_v, sem)
            d1 = pltpu.async_copy(
                table_hbm.at[cand_idx_v.at[pl.ds(s * CAND_PAD, 104)]],
                cand_rows_v.at[pl.ds(0, 104)], sem)
            d2 = pltpu.async_copy(
                table_hbm.at[cand_idx_v.at[pl.ds(s * CAND_PAD + 104, 104)]],
                cand_rows_v.at[pl.ds(104, 104)], sem)
            d3 = pltpu.async_copy(
                table_hbm.at[cand_idx_v.at[pl.ds(s * CAND_PAD + 208, 104)]],
                cand_rows_v.at[pl.ds(208, 104)], sem)
            d0.wait()
            d1.wait()
            d2.wait()
            d3.wait()

            def ctx_red(i, accs):
                return tuple(
                    accs[ch] + ctx_rows_v[i, pl.ds(ch * 16, 16)]
                    for ch in range(4))
            sums = lax.fori_loop(0, LCTX, ctx_red, (zero, zero, zero, zero))
            ctx_m = tuple(a * inv_ctx for a in sums)
            for ch in range(4):
                ctxe_v[s, pl.ds(ch * 16, 16)] = ctx_m[ch]

            def cand_body(cidx, _):
                r = cidx * LC
                for ch in range(4):
                    acc = cand_rows_v[r, pl.ds(ch * 16, 16)]
                    for k in range(1, LC):
                        acc = acc + cand_rows_v[r + k, pl.ds(ch * 16, 16)]
                    comb_v[pl.ds(cidx * EMB + ch * 16, 16)] = (
                        ctx_m[ch] + acc * inv_lc)
                return 0
            lax.fori_loop(0, C, cand_body, 0)

            pltpu.sync_copy(
                comb_v, comb_hbm.at[pl.ds((s0 + s) * C * EMB, C * EMB)])
            return carry

        lax.fori_loop(0, SPW, sample_body, 0)
        pltpu.sync_copy(ctxe_v, ctxe_hbm.at[pl.ds(s0, SPW)])

    return sc_kernel(table, ctx_idx_flat, cand_idx_flat)


def _tc_head(x, W, b, W2, b2, out_dim, block_rows):
    N = x.shape[0]
    H = W.shape[1]
    grid = N // block_rows

    def body(x_ref, w_ref, b_ref, w2_ref, b2_ref, o_ref):
        h = jnp.dot(x_ref[...], w_ref[...],
                    preferred_element_type=jnp.float32) + b_ref[...]
        h = jnp.maximum(h, 0.0)
        o_ref[...] = jnp.dot(h, w2_ref[...],
                             preferred_element_type=jnp.float32) + b2_ref[...]

    return pl.pallas_call(
        body,
        grid=(grid,),
        in_specs=[
            pl.BlockSpec((block_rows, EMB), lambda i: (i, 0)),
            pl.BlockSpec((EMB, H), lambda i: (0, 0)),
            pl.BlockSpec((1, H), lambda i: (0, 0)),
            pl.BlockSpec((H, out_dim), lambda i: (0, 0)),
            pl.BlockSpec((1, out_dim), lambda i: (0, 0)),
        ],
        out_specs=pl.BlockSpec((block_rows, out_dim), lambda i: (i, 0)),
        out_shape=jax.ShapeDtypeStruct((N, out_dim), jnp.float32),
    )(x, W, b, W2, b2)


def kernel(table, W1, b1, W2, b2, Wc1, bc1, Wc2, bc2,
           context_indices, candidate_indices):
    B, LCTX = context_indices.shape
    _, C, LC = candidate_indices.shape

    ctx_i = context_indices.astype(jnp.int32)
    cand_i = candidate_indices.astype(jnp.int32).reshape(B, C * LC)
    ctx_p = jnp.pad(ctx_i, ((0, 0), (0, CTX_PAD - LCTX))).reshape(-1)
    cand_p = jnp.pad(cand_i, ((0, 0), (0, CAND_PAD - C * LC))).reshape(-1)

    combined, ctx_emb = _sc_pool(table, ctx_p, cand_p, B, C, LCTX, LC)
    combined = combined.reshape(B * C, EMB)

    term_logits = _tc_head(ctx_emb, W1, b1.reshape(1, -1),
                           W2, b2.reshape(1, -1), 2, B)
    scores = _tc_head(combined, Wc1, bc1.reshape(1, -1),
                      Wc2, bc2.reshape(1, -1), 1, 2048)
    return term_logits, scores.reshape(B, C)


# speedup vs baseline: 2.0724x; 1.8276x over previous
import functools

import jax
import jax.numpy as jnp
from jax import lax
from jax.experimental import pallas as pl
from jax.experimental.pallas import tpu as pltpu
from jax.experimental.pallas import tpu_sc as plsc

EMB = 64
NW = 32
CAND_CHUNKS = ((0, 104), (104, 104), (208, 92))


def _sc_pool(table, ctx_idx, cand_idx, B, C, LCTX, LC):
    SPW = B // NW
    NCI = C * LC
    CE = C * EMB
    mesh = plsc.VectorSubcoreMesh(core_axis_name="c", subcore_axis_name="s")

    @functools.partial(
        pl.kernel,
        mesh=mesh,
        compiler_params=pltpu.CompilerParams(use_tc_tiling_on_sc=False),
        out_type=[
            jax.ShapeDtypeStruct((B * CE,), jnp.float32),
            jax.ShapeDtypeStruct((B, EMB), jnp.float32),
        ],
        scratch_types=[
            pltpu.VMEM((SPW, LCTX), jnp.int32),
            pltpu.VMEM((SPW, NCI), jnp.int32),
            pltpu.VMEM((2, LCTX, EMB), jnp.float32),
            pltpu.VMEM((2, NCI, EMB), jnp.float32),
            pltpu.VMEM((2, CE), jnp.float32),
            pltpu.VMEM((SPW, EMB), jnp.float32),
            pltpu.SemaphoreType.DMA,
            pltpu.SemaphoreType.DMA,
            pltpu.SemaphoreType.DMA,
            pltpu.SemaphoreType.DMA,
        ],
    )
    def sc_kernel(table_hbm, ctx_idx_hbm, cand_idx_hbm,
                  comb_hbm, ctxe_hbm,
                  ctx_idx_v, cand_idx_v, ctx_rows_v, cand_rows_v,
                  comb_v, ctxe_v, gsem0, gsem1, wsem0, wsem1):
        wid = lax.axis_index("s") * 2 + lax.axis_index("c")
        s0 = wid * SPW
        gsems = (gsem0, gsem1)
        wsems = (wsem0, wsem1)

        pltpu.sync_copy(ctx_idx_hbm.at[pl.ds(s0, SPW)], ctx_idx_v)
        pltpu.sync_copy(cand_idx_hbm.at[pl.ds(s0, SPW)], cand_idx_v)

        inv_ctx = jnp.float32(1.0 / LCTX)
        inv_lc = jnp.float32(1.0 / LC)
        zero = jnp.zeros((16,), jnp.float32)

        def gather_descs(s, b):
            descs = [pltpu.make_async_copy(
                table_hbm.at[ctx_idx_v.at[s]], ctx_rows_v.at[b], gsems[b])]
            for (off, ln) in CAND_CHUNKS:
                descs.append(pltpu.make_async_copy(
                    table_hbm.at[cand_idx_v.at[s, pl.ds(off, ln)]],
                    cand_rows_v.at[b, pl.ds(off, ln)], gsems[b]))
            return descs

        def issue(s, b):
            for d in gather_descs(s, b):
                d.start()

        def drain(s, b):
            for d in gather_descs(s, b):
                d.wait()

        def wdesc(s, b):
            return pltpu.make_async_copy(
                comb_v.at[b], comb_hbm.at[pl.ds((s0 + s) * CE, CE)], wsems[b])

        issue(0, 0)
        issue(1, 1)

        def step(s, b):
            drain(s, b)

            @pl.when(s >= 2)
            def _():
                wdesc(s - 2, b).wait()

            rows_c = ctx_rows_v.at[b]
            rows_k = cand_rows_v.at[b]
            out_c = comb_v.at[b]

            def ctx_red(i, accs):
                return tuple(
                    accs[ch] + rows_c[i, pl.ds(ch * 16, 16)]
                    for ch in range(4))
            sums = lax.fori_loop(0, LCTX, ctx_red, (zero, zero, zero, zero))
            ctx_m = tuple(a * inv_ctx for a in sums)
            for ch in range(4):
                ctxe_v[s, pl.ds(ch * 16, 16)] = ctx_m[ch]

            def cand_body(t, _):
                for u in range(2):
                    cidx = t * 2 + u
                    r = cidx * LC
                    for ch in range(4):
                        acc = rows_k[r, pl.ds(ch * 16, 16)]
                        for k in range(1, LC):
                            acc = acc + rows_k[r + k, pl.ds(ch * 16, 16)]
                        out_c[pl.ds(cidx * EMB + ch * 16, 16)] = (
                            ctx_m[ch] + acc * inv_lc)
                return 0
            lax.fori_loop(0, C // 2, cand_body, 0)

            wdesc(s, b).start()

            @pl.when(s + 2 < SPW)
            def _():
                issue(s + 2, b)

        def loop_body(t, carry):
            step(t * 2, 0)
            step(t * 2 + 1, 1)
            return carry

        lax.fori_loop(0, SPW // 2, loop_body, 0)

        wdesc(SPW - 2, 0).wait()
        wdesc(SPW - 1, 1).wait()

        pltpu.sync_copy(ctxe_v, ctxe_hbm.at[pl.ds(s0, SPW)])

    return sc_kernel(table, ctx_idx, cand_idx)


def _tc_head(x, W, b, W2, b2, out_dim, block_rows):
    N = x.shape[0]
    H = W.shape[1]
    grid = N // block_rows

    def body(x_ref, w_ref, b_ref, w2_ref, b2_ref, o_ref):
        h = jnp.dot(x_ref[...], w_ref[...],
                    preferred_element_type=jnp.float32) + b_ref[...]
        h = jnp.maximum(h, 0.0)
        o_ref[...] = jnp.dot(h, w2_ref[...],
                             preferred_element_type=jnp.float32) + b2_ref[...]

    return pl.pallas_call(
        body,
        grid=(grid,),
        in_specs=[
            pl.BlockSpec((block_rows, EMB), lambda i: (i, 0)),
            pl.BlockSpec((EMB, H), lambda i: (0, 0)),
            pl.BlockSpec((1, H), lambda i: (0, 0)),
            pl.BlockSpec((H, out_dim), lambda i: (0, 0)),
            pl.BlockSpec((1, out_dim), lambda i: (0, 0)),
        ],
        out_specs=pl.BlockSpec((block_rows, out_dim), lambda i: (i, 0)),
        out_shape=jax.ShapeDtypeStruct((N, out_dim), jnp.float32),
    )(x, W, b, W2, b2)


def kernel(table, W1, b1, W2, b2, Wc1, bc1, Wc2, bc2,
           context_indices, candidate_indices):
    B, LCTX = context_indices.shape
    _, C, LC = candidate_indices.shape

    ctx_i = context_indices.astype(jnp.int32)
    cand_i = candidate_indices.astype(jnp.int32).reshape(B, C * LC)

    combined, ctx_emb = _sc_pool(table, ctx_i, cand_i, B, C, LCTX, LC)
    combined = combined.reshape(B * C, EMB)

    term_logits = _tc_head(ctx_emb, W1, b1.reshape(1, -1),
                           W2, b2.reshape(1, -1), 2, B)
    scores = _tc_head(combined, Wc1, bc1.reshape(1, -1),
                      Wc2, bc2.reshape(1, -1), 1, 2048)
    return term_logits, scores.reshape(B, C)


# speedup vs baseline: 2.3608x; 1.1392x over previous
import functools

import jax
import jax.numpy as jnp
from jax import lax
from jax.experimental import pallas as pl
from jax.experimental.pallas import tpu as pltpu
from jax.experimental.pallas import tpu_sc as plsc

EMB = 64
NW = 32
CAND_CHUNKS = ((0, 104), (104, 104), (208, 92))


def _sc_pool(table, ctx_idx, cand_idx, B, C, LCTX, LC):
    SPW = B // NW
    NCI = C * LC
    PC = 2 * C
    mesh = plsc.VectorSubcoreMesh(core_axis_name="c", subcore_axis_name="s")

    @functools.partial(
        pl.kernel,
        mesh=mesh,
        compiler_params=pltpu.CompilerParams(use_tc_tiling_on_sc=False,
                                             needs_layout_passes=False),
        out_type=[
            jax.ShapeDtypeStruct((EMB, B * C), jnp.float32),
            jax.ShapeDtypeStruct((B, EMB), jnp.float32),
        ],
        scratch_types=[
            pltpu.VMEM((SPW, LCTX), jnp.int32),
            pltpu.VMEM((SPW, NCI), jnp.int32),
            pltpu.VMEM((2, LCTX, EMB), jnp.float32),
            pltpu.VMEM((2, NCI, EMB), jnp.float32),
            pltpu.VMEM((2, EMB, PC), jnp.float32),
            pltpu.VMEM((SPW, EMB), jnp.float32),
            pltpu.SemaphoreType.DMA,
            pltpu.SemaphoreType.DMA,
            pltpu.SemaphoreType.DMA,
            pltpu.SemaphoreType.DMA,
        ],
    )
    def sc_kernel(table_hbm, ctx_idx_hbm, cand_idx_hbm,
                  comb_hbm, ctxe_hbm,
                  ctx_idx_v, cand_idx_v, ctx_rows_v, cand_rows_v,
                  comb_v, ctxe_v, gsem0, gsem1, wsem0, wsem1):
        wid = lax.axis_index("s") * 2 + lax.axis_index("c")
        s0 = wid * SPW
        gsems = (gsem0, gsem1)
        wsems = (wsem0, wsem1)

        pltpu.sync_copy(ctx_idx_hbm.at[pl.ds(s0, SPW)], ctx_idx_v)
        pltpu.sync_copy(cand_idx_hbm.at[pl.ds(s0, SPW)], cand_idx_v)

        inv_ctx = jnp.float32(1.0 / LCTX)
        inv_lc = jnp.float32(1.0 / LC)
        zero = jnp.zeros((16,), jnp.float32)
        iota = lax.iota(jnp.int32, 16)
        riota = tuple(iota + 16 * ch for ch in range(4))

        def gather_descs(s, gb):
            descs = [pltpu.make_async_copy(
                table_hbm.at[ctx_idx_v.at[s]], ctx_rows_v.at[gb], gsems[gb])]
            for (off, ln) in CAND_CHUNKS:
                descs.append(pltpu.make_async_copy(
                    table_hbm.at[cand_idx_v.at[s, pl.ds(off, ln)]],
                    cand_rows_v.at[gb, pl.ds(off, ln)], gsems[gb]))
            return descs

        def issue(s, gb):
            for d in gather_descs(s, gb):
                d.start()

        def drain(s, gb):
            for d in gather_descs(s, gb):
                d.wait()

        def wdesc(p, wb):
            return pltpu.make_async_copy(
                comb_v.at[wb],
                comb_hbm.at[pl.ds(0, EMB), pl.ds((s0 + 2 * p) * C, PC)],
                wsems[wb])

        issue(0, 0)
        issue(1, 1)

        def pool_sample(s, gb, wb, u):
            drain(s, gb)

            rows_c = ctx_rows_v.at[gb]
            rows_k = cand_rows_v.at[gb]
            out_c = comb_v.at[wb]

            def ctx_red(i, accs):
                return tuple(
                    accs[ch] + rows_c[i, pl.ds(ch * 16, 16)]
                    for ch in range(4))
            sums = lax.fori_loop(0, LCTX, ctx_red, (zero, zero, zero, zero))
            ctx_m = tuple(a * inv_ctx for a in sums)
            for ch in range(4):
                ctxe_v[s, pl.ds(ch * 16, 16)] = ctx_m[ch]

            def cand_body(t, _):
                for v in range(2):
                    cidx = t * 2 + v
                    r = cidx * LC
                    col = jnp.full((16,), u * C + cidx, jnp.int32)
                    for ch in range(4):
                        acc = rows_k[r, pl.ds(ch * 16, 16)]
                        for k in range(1, LC):
                            acc = acc + rows_k[r + k, pl.ds(ch * 16, 16)]
                        plsc.store_scatter(out_c, [riota[ch], col],
                                           ctx_m[ch] + acc * inv_lc)
                return 0
            lax.fori_loop(0, C // 2, cand_body, 0)

            @pl.when(s + 2 < SPW)
            def _():
                issue(s + 2, gb)

        def loop_body(tt, carry):
            for wb in range(2):
                p = tt * 2 + wb
                s = p * 2

                @pl.when(p >= 2)
                def _():
                    wdesc(p - 2, wb).wait()

                pool_sample(s, 0, wb, 0)
                pool_sample(s + 1, 1, wb, 1)
                wdesc(p, wb).start()
            return carry

        lax.fori_loop(0, SPW // 4, loop_body, 0)

        wdesc(SPW // 2 - 2, 0).wait()
        wdesc(SPW // 2 - 1, 1).wait()

        pltpu.sync_copy(ctxe_v, ctxe_hbm.at[pl.ds(s0, SPW)])

    return sc_kernel(table, ctx_idx, cand_idx)


def _tc_term_head(x, W, b, W2, b2):
    N = x.shape[0]
    H = W.shape[1]
    OD = W2.shape[1]

    def body(x_ref, w_ref, b_ref, w2_ref, b2_ref, o_ref):
        h = jnp.dot(x_ref[...], w_ref[...],
                    preferred_element_type=jnp.float32) + b_ref[...]
        h = jnp.maximum(h, 0.0)
        o_ref[...] = jnp.dot(h, w2_ref[...],
                             preferred_element_type=jnp.float32) + b2_ref[...]

    return pl.pallas_call(
        body,
        grid=(1,),
        in_specs=[
            pl.BlockSpec((N, EMB), lambda i: (0, 0)),
            pl.BlockSpec((EMB, H), lambda i: (0, 0)),
            pl.BlockSpec((1, H), lambda i: (0, 0)),
            pl.BlockSpec((H, OD), lambda i: (0, 0)),
            pl.BlockSpec((1, OD), lambda i: (0, 0)),
        ],
        out_specs=pl.BlockSpec((N, OD), lambda i: (0, 0)),
        out_shape=jax.ShapeDtypeStruct((N, OD), jnp.float32),
    )(x, W, b, W2, b2)


def _tc_cand_head(xT, Wc1T, bc1c, Wc2T, bc2, block_cols):
    N = xT.shape[1]
    H = Wc1T.shape[0]
    grid = N // block_cols

    def body(x_ref, w1_ref, b1_ref, w2_ref, b2_ref, o_ref):
        h = jnp.dot(w1_ref[...], x_ref[...],
                    preferred_element_type=jnp.float32) + b1_ref[...]
        h = jnp.maximum(h, 0.0)
        s = jnp.dot(w2_ref[...], h,
                    preferred_element_type=jnp.float32) + b2_ref[...]
        o_ref[...] = s[None]

    return pl.pallas_call(
        body,
        grid=(grid,),
        in_specs=[
            pl.BlockSpec((EMB, block_cols), lambda i: (0, i)),
            pl.BlockSpec((H, EMB), lambda i: (0, 0)),
            pl.BlockSpec((H, 1), lambda i: (0, 0)),
            pl.BlockSpec((1, H), lambda i: (0, 0)),
            pl.BlockSpec((1, 1), lambda i: (0, 0)),
        ],
        out_specs=pl.BlockSpec((1, 1, block_cols), lambda i: (i, 0, 0)),
        out_shape=jax.ShapeDtypeStruct((grid, 1, block_cols), jnp.float32),
    )(xT, Wc1T, bc1c, Wc2T, bc2)


def kernel(table, W1, b1, W2, b2, Wc1, bc1, Wc2, bc2,
           context_indices, candidate_indices):
    B, LCTX = context_indices.shape
    _, C, LC = candidate_indices.shape

    ctx_i = context_indices.astype(jnp.int32)
    cand_i = candidate_indices.astype(jnp.int32).reshape(B, C * LC)

    combT, ctx_emb = _sc_pool(table, ctx_i, cand_i, B, C, LCTX, LC)

    term_logits = _tc_term_head(ctx_emb, W1, b1.reshape(1, -1),
                                W2, b2.reshape(1, -1))
    scores = _tc_cand_head(combT, Wc1.T, bc1.reshape(-1, 1),
                           Wc2.reshape(1, -1), bc2.reshape(1, 1), 2048)
    return term_logits, scores.reshape(B, C)
